# baseline (speedup 1.0000x reference)
import jax
import jax.numpy as jnp
from jax.experimental import pallas as pl
from jax.experimental.pallas import tpu as pltpu

_BM = 256


def _fused_block(x_ref, a0_ref, a1_ref, o_ref):
    t0 = jnp.dot(a0_ref[...], x_ref[...], preferred_element_type=jnp.float32)
    o_ref[0:_BM, :] = (t0 > 0.5).astype(jnp.float32)
    t1 = jnp.dot(a1_ref[...], x_ref[...], preferred_element_type=jnp.float32)
    o_ref[_BM : 2 * _BM, :] = (t1 > 0.5).astype(jnp.float32)


def kernel(x, a):
    m, k = a.shape
    n = x.shape[1]
    steps = m // (2 * _BM)
    return pl.pallas_call(
        _fused_block,
        grid=(steps,),
        in_specs=[
            pl.BlockSpec((k, n), lambda i: (0, 0)),
            pl.BlockSpec((_BM, k), lambda i: (2 * i, 0)),
            pl.BlockSpec((_BM, k), lambda i: (2 * i + 1, 0)),
        ],
        out_specs=pl.BlockSpec((2 * _BM, n), lambda i: (i, 0)),
        out_shape=jax.ShapeDtypeStruct((m, n), jnp.float32),
        compiler_params=pltpu.CompilerParams(
            dimension_semantics=("arbitrary",),
        ),
    )(x, a, a)
